# baseline (speedup 1.0000x reference)
import functools

import jax
import jax.numpy as jnp
from jax import lax
from jax.experimental import pallas as pl
from jax.experimental.pallas import tpu as pltpu
from jax.experimental.pallas import tpu_sc as plsc

VOCAB = 100000
EMB = 128
HID = 128
L = 50
B = 32

_NW = 32
_N_IDX = L * B
_N_PAD = 2048
_BPW = _N_PAD // _NW

_VT = 2000
_NV = VOCAB // _VT


def _sc_gather(table, idx_pad):
    mesh = plsc.VectorSubcoreMesh(core_axis_name="c", subcore_axis_name="s")

    @functools.partial(
        pl.kernel,
        mesh=mesh,
        out_type=jax.ShapeDtypeStruct((_N_PAD, EMB), jnp.float32),
        scratch_types=[
            pltpu.VMEM((_BPW,), jnp.int32),
            pltpu.VMEM((_BPW, EMB), jnp.float32),
            pltpu.SemaphoreType.DMA,
        ],
    )
    def gather_kernel(table_hbm, idx_hbm, out_hbm, idx_v, rows_v, sem):
        wid = lax.axis_index("s") * 2 + lax.axis_index("c")
        base = wid * _BPW
        pltpu.sync_copy(idx_hbm.at[pl.ds(base, _BPW)], idx_v)
        pltpu.async_copy(table_hbm.at[idx_v], rows_v, sem).wait()
        pltpu.sync_copy(rows_v, out_hbm.at[pl.ds(base, _BPW)])

    return gather_kernel(table, idx_pad)


def _rnn_body(emb_ref, wih_ref, whh_ref, bias_ref, hsT_ref):
    wih = wih_ref[...]
    whh = whh_ref[...]
    bias = bias_ref[...]

    def step(t, hT):
        x = emb_ref[t]
        pre = (
            lax.dot_general(wih, x, (((1,), (1,)), ((), ())))
            + lax.dot_general(whh, hT, (((1,), (0,)), ((), ())))
            + bias
        )
        hT_new = jnp.tanh(pre)
        hsT_ref[t] = hT_new
        return hT_new

    lax.fori_loop(0, L, step, jnp.zeros((HID, B), jnp.float32))


def _proj_body(wout_ref, hsT_ref, bout_ref, out_ref):
    w = wout_ref[...]
    h = hsT_ref[0]
    out_ref[0] = lax.dot_general(w, h, (((1,), (0,)), ((), ()))) + bout_ref[...]


def kernel(input_sequence, table, W_ih, W_hh, b_ih, b_hh, W_out, b_out):
    idx = input_sequence.reshape(-1).astype(jnp.int32)
    idx_pad = jnp.concatenate(
        [idx, jnp.zeros((_N_PAD - _N_IDX,), jnp.int32)]
    )
    emb = _sc_gather(table, idx_pad)
    emb3 = emb[:_N_IDX].reshape(L, B, EMB)
    bias_col = (b_ih + b_hh).reshape(HID, 1)

    hsT = pl.pallas_call(
        _rnn_body,
        out_shape=jax.ShapeDtypeStruct((L, HID, B), jnp.float32),
    )(emb3, W_ih, W_hh, bias_col)

    out = pl.pallas_call(
        _proj_body,
        grid=(_NV, L),
        in_specs=[
            pl.BlockSpec((_VT, HID), lambda v, l: (v, 0)),
            pl.BlockSpec((1, HID, B), lambda v, l: (l, 0, 0)),
            pl.BlockSpec((_VT, 1), lambda v, l: (v, 0)),
        ],
        out_specs=pl.BlockSpec((1, _VT, B), lambda v, l: (l, v, 0)),
        out_shape=jax.ShapeDtypeStruct((L, VOCAB, B), jnp.float32),
    )(W_out, hsT, b_out.reshape(VOCAB, 1))
    return out


# speedup vs baseline: 1.5729x; 1.5729x over previous
import functools

import jax
import jax.numpy as jnp
from jax import lax
from jax.experimental import pallas as pl
from jax.experimental.pallas import tpu as pltpu
from jax.experimental.pallas import tpu_sc as plsc

VOCAB = 100000
EMB = 128
HID = 128
L = 50
B = 32

_NW = 32
_N_IDX = L * B
_N_PAD = 2048
_BPW = _N_PAD // _NW

_VT = 20000
_NV = VOCAB // _VT
_XT = _VT // 4


def _sc_gather(table, idx_pad):
    mesh = plsc.VectorSubcoreMesh(core_axis_name="c", subcore_axis_name="s")

    @functools.partial(
        pl.kernel,
        mesh=mesh,
        out_type=jax.ShapeDtypeStruct((_N_PAD, EMB), jnp.float32),
        scratch_types=[
            pltpu.VMEM((_BPW,), jnp.int32),
            pltpu.VMEM((_BPW, EMB), jnp.float32),
            pltpu.SemaphoreType.DMA,
        ],
    )
    def gather_kernel(table_hbm, idx_hbm, out_hbm, idx_v, rows_v, sem):
        wid = lax.axis_index("s") * 2 + lax.axis_index("c")
        base = wid * _BPW
        pltpu.sync_copy(idx_hbm.at[pl.ds(base, _BPW)], idx_v)
        pltpu.async_copy(table_hbm.at[idx_v], rows_v, sem).wait()
        pltpu.sync_copy(rows_v, out_hbm.at[pl.ds(base, _BPW)])

    return gather_kernel(table, idx_pad)


def _rnn_body(emb_ref, wih_ref, whh_ref, bias_ref, hsT_ref):
    wih = wih_ref[...]
    whh = whh_ref[...]
    bias = bias_ref[...]

    def step(t, hT):
        x = emb_ref[t]
        pre = (
            lax.dot_general(wih, x, (((1,), (1,)), ((), ())))
            + lax.dot_general(whh, hT, (((1,), (0,)), ((), ())))
            + bias
        )
        hT_new = jnp.tanh(pre)
        hsT_ref[t] = hT_new
        return hT_new

    lax.fori_loop(0, L, step, jnp.zeros((HID, B), jnp.float32))


def _proj_body(a_ref, c_ref, bias_ref, out_ref):
    l = pl.program_id(1)
    p = lax.dot_general(
        a_ref[...], c_ref[l],
        (((1,), (0,)), ((), ())),
        preferred_element_type=jnp.float32,
    )
    out_ref[0] = p + bias_ref[...]


def kernel(input_sequence, table, W_ih, W_hh, b_ih, b_hh, W_out, b_out):
    idx = input_sequence.reshape(-1).astype(jnp.int32)
    idx_pad = jnp.concatenate(
        [idx, jnp.zeros((_N_PAD - _N_IDX,), jnp.int32)]
    )
    emb = _sc_gather(table, idx_pad)
    emb3 = emb[:_N_IDX].reshape(L, B, EMB)
    bias_col = (b_ih + b_hh).reshape(HID, 1)

    hsT = pl.pallas_call(
        _rnn_body,
        out_shape=jax.ShapeDtypeStruct((L, HID, B), jnp.float32),
    )(emb3, W_ih, W_hh, bias_col)

    a_packed = W_out.reshape(VOCAB // 4, 4 * HID).astype(jnp.bfloat16)
    eye4 = jnp.eye(4, dtype=hsT.dtype)
    c_all = (eye4[:, None, :, None] * hsT[:, None, :, None, :]).reshape(
        L, 4 * HID, 4 * B
    ).astype(jnp.bfloat16)
    bias_packed = jnp.repeat(b_out.reshape(VOCAB // 4, 4), B, axis=1)

    out_packed = pl.pallas_call(
        _proj_body,
        grid=(_NV, L),
        in_specs=[
            pl.BlockSpec((_XT, 4 * HID), lambda v, l: (v, 0)),
            pl.BlockSpec((L, 4 * HID, 4 * B), lambda v, l: (0, 0, 0)),
            pl.BlockSpec((_XT, 4 * B), lambda v, l: (v, 0)),
        ],
        out_specs=pl.BlockSpec((1, _XT, 4 * B), lambda v, l: (l, v, 0)),
        out_shape=jax.ShapeDtypeStruct((L, VOCAB // 4, 4 * B), jnp.float32),
    )(a_packed, c_all, bias_packed)
    return out_packed.reshape(L, VOCAB, B)


# speedup vs baseline: 14.1073x; 8.9689x over previous
import functools

import jax
import jax.numpy as jnp
from jax import lax
from jax.experimental import pallas as pl
from jax.experimental.pallas import tpu as pltpu
from jax.experimental.pallas import tpu_sc as plsc

VOCAB = 100000
EMB = 128
HID = 128
L = 50
B = 32

_NW = 32
_N_IDX = L * B
_N_PAD = 2048
_BPW = _N_PAD // _NW

_VT = 2048
_NV = (VOCAB + _VT - 1) // _VT


def _sc_gather(table, idx_pad):
    mesh = plsc.VectorSubcoreMesh(core_axis_name="c", subcore_axis_name="s")

    @functools.partial(
        pl.kernel,
        mesh=mesh,
        out_type=jax.ShapeDtypeStruct((_N_PAD, EMB), jnp.float32),
        scratch_types=[
            pltpu.VMEM((_BPW,), jnp.int32),
            pltpu.VMEM((_BPW, EMB), jnp.float32),
            pltpu.SemaphoreType.DMA,
        ],
    )
    def gather_kernel(table_hbm, idx_hbm, out_hbm, idx_v, rows_v, sem):
        wid = lax.axis_index("s") * 2 + lax.axis_index("c")
        base = wid * _BPW
        pltpu.sync_copy(idx_hbm.at[pl.ds(base, _BPW)], idx_v)
        pltpu.async_copy(table_hbm.at[idx_v], rows_v, sem).wait()
        pltpu.sync_copy(rows_v, out_hbm.at[pl.ds(base, _BPW)])

    return gather_kernel(table, idx_pad)


def _rnn_body(emb_ref, wih_ref, whh_ref, bias_ref, hs_ref):
    wih = wih_ref[...]
    whh = whh_ref[...]
    bias = bias_ref[...]

    def step(t, h):
        x = emb_ref[t]
        pre = (
            lax.dot_general(x, wih, (((1,), (1,)), ((), ())))
            + lax.dot_general(h, whh, (((1,), (1,)), ((), ())))
            + bias
        )
        h_new = jnp.tanh(pre)
        hs_ref[t] = h_new
        return h_new

    lax.fori_loop(0, L, step, jnp.zeros((B, HID), jnp.float32))


def _proj_body(hs_ref, wout_ref, bias_ref, out_ref):
    w = wout_ref[...].astype(jnp.bfloat16)
    r = lax.dot_general(
        hs_ref[...], w,
        (((1,), (1,)), ((), ())),
        preferred_element_type=jnp.float32,
    )
    out_ref[...] = r + bias_ref[...]


def kernel(input_sequence, table, W_ih, W_hh, b_ih, b_hh, W_out, b_out):
    idx = input_sequence.reshape(-1).astype(jnp.int32)
    idx_pad = jnp.concatenate(
        [idx, jnp.zeros((_N_PAD - _N_IDX,), jnp.int32)]
    )
    emb = _sc_gather(table, idx_pad)
    emb3 = emb[:_N_IDX].reshape(L, B, EMB)
    bias_row = (b_ih + b_hh).reshape(1, HID)

    hs = pl.pallas_call(
        _rnn_body,
        out_shape=jax.ShapeDtypeStruct((L, B, HID), jnp.float32),
    )(emb3, W_ih, W_hh, bias_row)

    hs_bf = hs.reshape(L * B, HID).astype(jnp.bfloat16)

    out2d = pl.pallas_call(
        _proj_body,
        grid=(_NV,),
        in_specs=[
            pl.BlockSpec((L * B, HID), lambda v: (0, 0)),
            pl.BlockSpec((_VT, HID), lambda v: (v, 0)),
            pl.BlockSpec((1, _VT), lambda v: (0, v)),
        ],
        out_specs=pl.BlockSpec((L * B, _VT), lambda v: (0, v)),
        out_shape=jax.ShapeDtypeStruct((L * B, VOCAB), jnp.float32),
    )(hs_bf, W_out, b_out.reshape(1, VOCAB))

    return jnp.transpose(out2d.reshape(L, B, VOCAB), (0, 2, 1))


# speedup vs baseline: 14.3345x; 1.0161x over previous
import functools

import jax
import jax.numpy as jnp
from jax import lax
from jax.experimental import pallas as pl
from jax.experimental.pallas import tpu as pltpu
from jax.experimental.pallas import tpu_sc as plsc

VOCAB = 100000
EMB = 128
HID = 128
L = 50
B = 32

_NW = 32
_N_IDX = L * B
_N_PAD = 2048
_BPW = _N_PAD // _NW

_VT = 2048
_NV = (VOCAB + _VT - 1) // _VT


def _sc_gather(table, idx_pad):
    mesh = plsc.VectorSubcoreMesh(core_axis_name="c", subcore_axis_name="s")

    @functools.partial(
        pl.kernel,
        mesh=mesh,
        out_type=jax.ShapeDtypeStruct((_N_PAD, EMB), jnp.float32),
        scratch_types=[
            pltpu.VMEM((_BPW,), jnp.int32),
            pltpu.VMEM((_BPW, EMB), jnp.float32),
            pltpu.SemaphoreType.DMA,
        ],
    )
    def gather_kernel(table_hbm, idx_hbm, out_hbm, idx_v, rows_v, sem):
        wid = lax.axis_index("s") * 2 + lax.axis_index("c")
        base = wid * _BPW
        pltpu.sync_copy(idx_hbm.at[pl.ds(base, _BPW)], idx_v)
        pltpu.async_copy(table_hbm.at[idx_v], rows_v, sem).wait()
        pltpu.sync_copy(rows_v, out_hbm.at[pl.ds(base, _BPW)])

    return gather_kernel(table, idx_pad)


def _fused_body(emb_ref, wih_ref, whh_ref, bih_ref, bhh_ref,
                wout_ref, bout_ref, out_ref, hs_scr):
    @pl.when(pl.program_id(0) == 0)
    def _run_rnn():
        wih = wih_ref[...]
        whh = whh_ref[...]
        bias = bih_ref[...] + bhh_ref[...]

        def step(t, h):
            x = emb_ref[pl.ds(t * B, B), :]
            pre = (
                lax.dot_general(x, wih, (((1,), (1,)), ((), ())))
                + lax.dot_general(h, whh, (((1,), (1,)), ((), ())))
                + bias
            )
            h_new = jnp.tanh(pre)
            hs_scr[pl.ds(t * B, B), :] = h_new.astype(jnp.bfloat16)
            return h_new

        lax.fori_loop(0, L, step, jnp.zeros((B, HID), jnp.float32))

    w = wout_ref[...].astype(jnp.bfloat16)
    r = lax.dot_general(
        hs_scr[...], w,
        (((1,), (1,)), ((), ())),
        preferred_element_type=jnp.float32,
    )
    out_ref[...] = r + bout_ref[...]


def kernel(input_sequence, table, W_ih, W_hh, b_ih, b_hh, W_out, b_out):
    idx = input_sequence.reshape(-1).astype(jnp.int32)
    idx_pad = jnp.concatenate(
        [idx, jnp.zeros((_N_PAD - _N_IDX,), jnp.int32)]
    )
    emb = _sc_gather(table, idx_pad)

    out2d = pl.pallas_call(
        _fused_body,
        grid=(_NV,),
        in_specs=[
            pl.BlockSpec((_N_PAD, EMB), lambda v: (0, 0)),
            pl.BlockSpec((HID, EMB), lambda v: (0, 0)),
            pl.BlockSpec((HID, HID), lambda v: (0, 0)),
            pl.BlockSpec((1, HID), lambda v: (0, 0)),
            pl.BlockSpec((1, HID), lambda v: (0, 0)),
            pl.BlockSpec((_VT, HID), lambda v: (v, 0)),
            pl.BlockSpec((1, _VT), lambda v: (0, v)),
        ],
        out_specs=pl.BlockSpec((L * B, _VT), lambda v: (0, v)),
        out_shape=jax.ShapeDtypeStruct((L * B, VOCAB), jnp.float32),
        scratch_shapes=[pltpu.VMEM((L * B, HID), jnp.bfloat16)],
    )(emb, W_ih, W_hh, b_ih.reshape(1, HID), b_hh.reshape(1, HID),
      W_out, b_out.reshape(1, VOCAB))

    return jnp.transpose(out2d.reshape(L, B, VOCAB), (0, 2, 1))


# speedup vs baseline: 15.2910x; 1.0667x over previous
import functools

import jax
import jax.numpy as jnp
from jax import lax
from jax.experimental import pallas as pl
from jax.experimental.pallas import tpu as pltpu
from jax.experimental.pallas import tpu_sc as plsc

VOCAB = 100000
EMB = 128
HID = 128
L = 50
B = 32

_NW = 32
_N_IDX = L * B
_BPW = 64
_NW_USED = _N_IDX // _BPW

_VT = 2048
_NV = (VOCAB + _VT - 1) // _VT


def _sc_gather(table, idx):
    mesh = plsc.VectorSubcoreMesh(core_axis_name="c", subcore_axis_name="s")

    @functools.partial(
        pl.kernel,
        mesh=mesh,
        out_type=jax.ShapeDtypeStruct((_N_IDX, EMB), jnp.float32),
        scratch_types=[
            pltpu.VMEM((_BPW,), jnp.int32),
            pltpu.VMEM((_BPW, EMB), jnp.float32),
            pltpu.SemaphoreType.DMA,
        ],
    )
    def gather_kernel(table_hbm, idx_hbm, out_hbm, idx_v, rows_v, sem):
        wid = lax.axis_index("s") * 2 + lax.axis_index("c")

        @pl.when(wid < _NW_USED)
        def _do():
            base = wid * _BPW
            pltpu.sync_copy(idx_hbm.at[pl.ds(base, _BPW)], idx_v)
            pltpu.async_copy(table_hbm.at[idx_v], rows_v, sem).wait()
            pltpu.sync_copy(rows_v, out_hbm.at[pl.ds(base, _BPW)])

    return gather_kernel(table, idx)


def _fused_body(emb_ref, wih_ref, whh_ref, bih_ref, bhh_ref,
                wout_ref, bout_ref, out_ref, hs_scr):
    @pl.when(pl.program_id(0) == 0)
    def _run_rnn():
        wih = wih_ref[...]
        whh = whh_ref[...]
        bias = bih_ref[...] + bhh_ref[...]

        def step(t, h):
            x = emb_ref[pl.ds(t * B, B), :]
            pre = (
                lax.dot_general(x, wih, (((1,), (1,)), ((), ())))
                + lax.dot_general(h, whh, (((1,), (1,)), ((), ())))
                + bias
            )
            h_new = jnp.tanh(pre)
            hs_scr[pl.ds(t * B, B), :] = h_new.astype(jnp.bfloat16)
            return h_new

        lax.fori_loop(0, L, step, jnp.zeros((B, HID), jnp.float32))

    w = wout_ref[...].astype(jnp.bfloat16)
    r = lax.dot_general(
        hs_scr[...], w,
        (((1,), (1,)), ((), ())),
        preferred_element_type=jnp.float32,
    )
    out_ref[...] = r + bout_ref[...]


def kernel(input_sequence, table, W_ih, W_hh, b_ih, b_hh, W_out, b_out):
    idx = input_sequence.reshape(-1).astype(jnp.int32)
    emb = _sc_gather(table, idx)

    out2d = pl.pallas_call(
        _fused_body,
        grid=(_NV,),
        in_specs=[
            pl.BlockSpec((_N_IDX, EMB), lambda v: (0, 0)),
            pl.BlockSpec((HID, EMB), lambda v: (0, 0)),
            pl.BlockSpec((HID, HID), lambda v: (0, 0)),
            pl.BlockSpec((1, HID), lambda v: (0, 0)),
            pl.BlockSpec((1, HID), lambda v: (0, 0)),
            pl.BlockSpec((_VT, HID), lambda v: (v, 0)),
            pl.BlockSpec((1, _VT), lambda v: (0, v)),
        ],
        out_specs=pl.BlockSpec((L * B, _VT), lambda v: (0, v)),
        out_shape=jax.ShapeDtypeStruct((L * B, VOCAB), jnp.float32),
        scratch_shapes=[pltpu.VMEM((L * B, HID), jnp.bfloat16)],
    )(emb, W_ih, W_hh, b_ih.reshape(1, HID), b_hh.reshape(1, HID),
      W_out, b_out.reshape(1, VOCAB))

    return jnp.transpose(out2d.reshape(L, B, VOCAB), (0, 2, 1))


# speedup vs baseline: 15.5754x; 1.0186x over previous
import functools

import jax
import jax.numpy as jnp
from jax import lax
from jax.experimental import pallas as pl
from jax.experimental.pallas import tpu as pltpu
from jax.experimental.pallas import tpu_sc as plsc

VOCAB = 100000
EMB = 128
HID = 128
L = 50
B = 32

_NW = 32
_N_IDX = L * B
_BPW = 64
_NW_USED = _N_IDX // _BPW

_VT = 3072
_NV = (VOCAB + _VT - 1) // _VT


def _sc_gather(table, idx):
    mesh = plsc.VectorSubcoreMesh(core_axis_name="c", subcore_axis_name="s")

    @functools.partial(
        pl.kernel,
        mesh=mesh,
        out_type=jax.ShapeDtypeStruct((_N_IDX, EMB), jnp.float32),
        scratch_types=[
            pltpu.VMEM((_BPW,), jnp.int32),
            pltpu.VMEM((_BPW, EMB), jnp.float32),
            pltpu.SemaphoreType.DMA,
        ],
    )
    def gather_kernel(table_hbm, idx_hbm, out_hbm, idx_v, rows_v, sem):
        wid = lax.axis_index("s") * 2 + lax.axis_index("c")

        @pl.when(wid < _NW_USED)
        def _do():
            base = wid * _BPW
            pltpu.sync_copy(idx_hbm.at[pl.ds(base, _BPW)], idx_v)
            pltpu.async_copy(table_hbm.at[idx_v], rows_v, sem).wait()
            pltpu.sync_copy(rows_v, out_hbm.at[pl.ds(base, _BPW)])

    return gather_kernel(table, idx)


def _fused_body(emb_ref, wih_ref, whh_ref, bih_ref, bhh_ref,
                wout_ref, bout_ref, out_ref, hs_scr):
    @pl.when(pl.program_id(0) == 0)
    def _run_rnn():
        wih = wih_ref[...]
        whh = whh_ref[...]
        bias = bih_ref[...] + bhh_ref[...]

        def step(t, h):
            x = emb_ref[pl.ds(t * B, B), :]
            pre = (
                lax.dot_general(x, wih, (((1,), (1,)), ((), ())))
                + lax.dot_general(h, whh, (((1,), (1,)), ((), ())))
                + bias
            )
            h_new = jnp.tanh(pre)
            hs_scr[pl.ds(t * B, B), :] = h_new.astype(jnp.bfloat16)
            return h_new

        lax.fori_loop(0, L, step, jnp.zeros((B, HID), jnp.float32))

    w = wout_ref[...].astype(jnp.bfloat16)
    r = lax.dot_general(
        hs_scr[...], w,
        (((1,), (1,)), ((), ())),
        preferred_element_type=jnp.float32,
    )
    out_ref[...] = r + bout_ref[...]


def kernel(input_sequence, table, W_ih, W_hh, b_ih, b_hh, W_out, b_out):
    idx = input_sequence.reshape(-1).astype(jnp.int32)
    emb = _sc_gather(table, idx)

    out2d = pl.pallas_call(
        _fused_body,
        grid=(_NV,),
        in_specs=[
            pl.BlockSpec((_N_IDX, EMB), lambda v: (0, 0)),
            pl.BlockSpec((HID, EMB), lambda v: (0, 0)),
            pl.BlockSpec((HID, HID), lambda v: (0, 0)),
            pl.BlockSpec((1, HID), lambda v: (0, 0)),
            pl.BlockSpec((1, HID), lambda v: (0, 0)),
            pl.BlockSpec((_VT, HID), lambda v: (v, 0)),
            pl.BlockSpec((1, _VT), lambda v: (0, v)),
        ],
        out_specs=pl.BlockSpec((L * B, _VT), lambda v: (0, v)),
        out_shape=jax.ShapeDtypeStruct((L * B, VOCAB), jnp.float32),
        scratch_shapes=[pltpu.VMEM((L * B, HID), jnp.bfloat16)],
    )(emb, W_ih, W_hh, b_ih.reshape(1, HID), b_hh.reshape(1, HID),
      W_out, b_out.reshape(1, VOCAB))

    return jnp.transpose(out2d.reshape(L, B, VOCAB), (0, 2, 1))
